# baseline (speedup 1.0000x reference)
import functools

import jax
import jax.numpy as jnp
from jax import lax
from jax.experimental import pallas as pl
from jax.experimental.pallas import tpu as pltpu
from jax.experimental.pallas import tpu_sc as plsc

_B = 16384
_D = 64
_NC = 2
_NS = 16
_NW = _NC * _NS
_BPW = _B // _NW
_L = 16
_NG = _BPW // _L


def _encoder_body(row_idx_hbm, col_idx_hbm, dir_idx_hbm,
                  row_emb_hbm, col_emb_hbm, dir_emb_hbm, out_hbm,
                  ridx, cidx, didx, rbuf, cbuf, dirv, sem):
    wid = lax.axis_index("s") * _NC + lax.axis_index("c")
    base = wid * _BPW

    pltpu.sync_copy(row_idx_hbm.at[pl.ds(base, _BPW)], ridx)
    pltpu.sync_copy(col_idx_hbm.at[pl.ds(base, _BPW)], cidx)
    pltpu.sync_copy(dir_idx_hbm.at[pl.ds(base, _BPW)], didx)
    pltpu.sync_copy(dir_emb_hbm, dirv)

    def issue(g, _):
        rv = ridx[pl.ds(g * _L, _L)]
        cv = cidx[pl.ds(g * _L, _L)]
        for k in range(_L):
            j = g * _L + k
            pltpu.async_copy(row_emb_hbm.at[pl.ds(rv[k], 1)],
                             rbuf.at[pl.ds(j, 1)], sem)
            pltpu.async_copy(col_emb_hbm.at[pl.ds(cv[k], 1)],
                             cbuf.at[pl.ds(j, 1)], sem)
        return 0

    lax.fori_loop(0, _NG, issue, 0)

    def drain(g, _):
        for k in range(_L):
            j = g * _L + k
            pltpu.make_async_copy(row_emb_hbm.at[pl.ds(0, 1)],
                                  rbuf.at[pl.ds(j, 1)], sem).wait()
            pltpu.make_async_copy(col_emb_hbm.at[pl.ds(0, 1)],
                                  cbuf.at[pl.ds(j, 1)], sem).wait()
        return 0

    lax.fori_loop(0, _NG, drain, 0)

    nt = _D // _L
    d0 = [dirv[0, pl.ds(t * _L, _L)] for t in range(nt)]
    dd = [dirv[1, pl.ds(t * _L, _L)] - d0[t] for t in range(nt)]

    def combine(g, _):
        fv = didx[pl.ds(g * _L, _L)].astype(jnp.float32)
        for k in range(_L):
            b = g * _L + k
            f = fv[k]
            for t in range(nt):
                s = pl.ds(t * _L, _L)
                rbuf[b, s] = rbuf[b, s] + cbuf[b, s] + (d0[t] + f * dd[t])
        return 0

    lax.fori_loop(0, _NG, combine, 0)

    pltpu.sync_copy(rbuf, out_hbm.at[pl.ds(base, _BPW)])


_encoder = functools.partial(
    pl.kernel,
    out_type=jax.ShapeDtypeStruct((_B, _D), jnp.float32),
    mesh=plsc.VectorSubcoreMesh(core_axis_name="c", subcore_axis_name="s"),
    scratch_types=[
        pltpu.VMEM((_BPW,), jnp.int32),
        pltpu.VMEM((_BPW,), jnp.int32),
        pltpu.VMEM((_BPW,), jnp.int32),
        pltpu.VMEM((_BPW, _D), jnp.float32),
        pltpu.VMEM((_BPW, _D), jnp.float32),
        pltpu.VMEM((2, _D), jnp.float32),
        pltpu.SemaphoreType.DMA,
    ],
    compiler_params=pltpu.CompilerParams(use_tc_tiling_on_sc=False),
)(_encoder_body)


def kernel(row_idx, col_idx, dir_idx, row_emb, col_emb, dir_emb):
    ri = row_idx.astype(jnp.int32)
    ci = col_idx.astype(jnp.int32)
    di = dir_idx.astype(jnp.int32)
    return _encoder(ri, ci, di, row_emb, col_emb, dir_emb)


# speedup vs baseline: 1.3760x; 1.3760x over previous
import functools

import jax
import jax.numpy as jnp
from jax import lax
from jax.experimental import pallas as pl
from jax.experimental.pallas import tpu as pltpu
from jax.experimental.pallas import tpu_sc as plsc

_B = 16384
_D = 64
_NC = 2
_NS = 16
_NW = _NC * _NS
_BPW = _B // _NW
_L = 16
_NG = _BPW // _L
_HPW = _BPW // 2


def _encoder_body(row_idx_hbm, col_idx_hbm, dir_idx_hbm,
                  row_emb_hbm, col_emb_hbm, dir_emb_hbm, out_hbm,
                  ridx, cidx, didx, rbuf, cbuf, dirv, sem):
    wid = lax.axis_index("s") * _NC + lax.axis_index("c")
    base = wid * _BPW

    pltpu.sync_copy(row_idx_hbm.at[pl.ds(base, _BPW)], ridx)
    pltpu.sync_copy(col_idx_hbm.at[pl.ds(base, _BPW)], cidx)
    pltpu.sync_copy(dir_idx_hbm.at[pl.ds(base, _BPW)], didx)
    pltpu.sync_copy(dir_emb_hbm.at[pl.ds(0, 1)], dirv.at[pl.ds(0, 1)])
    pltpu.sync_copy(dir_emb_hbm.at[pl.ds(1, 1)], dirv.at[pl.ds(1, 1)])

    nt = _D // _L
    d0 = [dirv[0, pl.ds(t * _L, _L)] for t in range(nt)]
    dd = [dirv[1, pl.ds(t * _L, _L)] - d0[t] for t in range(nt)]

    for h in range(2):
        off = h * _HPW

        def issue(g, _):
            rv = ridx[pl.ds(off + g * _L, _L)]
            cv = cidx[pl.ds(off + g * _L, _L)]
            for k in range(_L):
                j = g * _L + k
                pltpu.async_copy(row_emb_hbm.at[pl.ds(rv[k], 1)],
                                 rbuf.at[pl.ds(j, 1)], sem)
                pltpu.async_copy(col_emb_hbm.at[pl.ds(cv[k], 1)],
                                 cbuf.at[pl.ds(j, 1)], sem)
            return 0

        lax.fori_loop(0, _HPW // _L, issue, 0)

        def drain(g, _):
            for k in range(_L):
                j = g * _L + k
                pltpu.make_async_copy(row_emb_hbm.at[pl.ds(0, 1)],
                                      rbuf.at[pl.ds(j, 1)], sem).wait()
                pltpu.make_async_copy(col_emb_hbm.at[pl.ds(0, 1)],
                                      cbuf.at[pl.ds(j, 1)], sem).wait()
            return 0

        lax.fori_loop(0, _HPW // _L, drain, 0)

        def combine(g, _):
            fv = didx[pl.ds(off + g * _L, _L)].astype(jnp.float32)
            for k in range(_L):
                b = g * _L + k
                f = fv[k]
                for t in range(nt):
                    s = pl.ds(t * _L, _L)
                    rbuf[b, s] = rbuf[b, s] + cbuf[b, s] + (d0[t] + f * dd[t])
            return 0

        lax.fori_loop(0, _HPW // _L, combine, 0)

        pltpu.sync_copy(rbuf, out_hbm.at[pl.ds(base + off, _HPW)])


_encoder = functools.partial(
    pl.kernel,
    out_type=jax.ShapeDtypeStruct((_B, _D), jnp.float32),
    mesh=plsc.VectorSubcoreMesh(core_axis_name="c", subcore_axis_name="s"),
    scratch_types=[
        pltpu.VMEM((_BPW,), jnp.int32),
        pltpu.VMEM((_BPW,), jnp.int32),
        pltpu.VMEM((_BPW,), jnp.int32),
        pltpu.VMEM((_HPW, _D), jnp.float32),
        pltpu.VMEM((_HPW, _D), jnp.float32),
        pltpu.VMEM((2, _D), jnp.float32),
        pltpu.SemaphoreType.DMA,
    ],
    compiler_params=pltpu.CompilerParams(use_tc_tiling_on_sc=True),
)(_encoder_body)


def kernel(row_idx, col_idx, dir_idx, row_emb, col_emb, dir_emb):
    ri = row_idx.astype(jnp.int32)
    ci = col_idx.astype(jnp.int32)
    di = dir_idx.astype(jnp.int32)
    return _encoder(ri, ci, di, row_emb, col_emb, dir_emb)
